# baseline (speedup 1.0000x reference)
import functools

import jax
import jax.numpy as jnp
from jax import lax
from jax.experimental import pallas as pl
from jax.experimental.pallas import tpu as pltpu
from jax.experimental.pallas import tpu_sc as plsc

NC = 2
NS = 16
C = 128
DEPTH = 3


def _seg_sum_halves(xact, packp, n, nchunk, shift):
    d2 = xact.shape[1] // 2
    rpt = -(-(-(-n // NS)) // 8) * 8
    last_rows = n - (NS - 1) * rpt
    last_zero = (n + 8) - (NS - 1) * rpt
    mesh = plsc.VectorSubcoreMesh(core_axis_name="c", subcore_axis_name="s")

    @functools.partial(
        pl.kernel,
        out_type=jax.ShapeDtypeStruct((NC, n, d2), jnp.float32),
        mesh=mesh,
        compiler_params=pltpu.CompilerParams(use_tc_tiling_on_sc=False),
        scratch_types=[
            pltpu.VMEM((nchunk // 2, C), jnp.int32),
            pltpu.VMEM((DEPTH, C), jnp.int32),
            pltpu.VMEM((DEPTH, C), jnp.int32),
            [pltpu.VMEM((C, d2), jnp.float32) for _ in range(DEPTH)],
            [pltpu.SemaphoreType.DMA for _ in range(DEPTH)],
            pltpu.VMEM_SHARED((n + 8, d2), jnp.float32),
            pltpu.VMEM_SHARED((n, d2), jnp.float32),
        ],
    )
    def seg(xact_hbm, packp_hbm, out_hbm,
            pidx, sring, dring, bufs, gsems, acc, xtab):
        c = lax.axis_index("c")
        s = lax.axis_index("s")

        zero = jnp.zeros((16,), jnp.float32)

        @pl.loop(0, C)
        def _(r):
            for col in range(d2 // 16):
                bufs[0][r, pl.ds(col * 16, 16)] = zero

        base = s * rpt

        def zero_acc(cnt):
            nfull, tail = cnt // C, cnt % C
            for k in range(nfull):
                pltpu.sync_copy(bufs[0], acc.at[pl.ds(base + k * C, C)])
            if tail:
                pltpu.sync_copy(bufs[0].at[pl.ds(0, tail)],
                                acc.at[pl.ds(base + nfull * C, tail)])

        @pl.when(s < NS - 1)
        def _():
            zero_acc(rpt)

        @pl.when(s == NS - 1)
        def _():
            zero_acc(last_zero)

        col = pl.ds(c * d2, d2)

        @pl.when(s < NS - 1)
        def _():
            pltpu.sync_copy(xact_hbm.at[pl.ds(base, rpt), col],
                            xtab.at[pl.ds(base, rpt)])

        @pl.when(s == NS - 1)
        def _():
            pltpu.sync_copy(xact_hbm.at[pl.ds(base, last_rows), col],
                            xtab.at[pl.ds(base, last_rows)])

        mask = jnp.full((16,), (1 << shift) - 1, jnp.int32)

        def unpack(j, slot):
            for col in range(C // 16):
                sl = pl.ds(col * 16, 16)
                v = pidx[j, sl]
                dring[slot, sl] = lax.shift_right_logical(v, shift)
                sring[slot, sl] = lax.bitwise_and(v, mask)

        def gather(b):
            pltpu.async_copy(xtab.at[sring.at[b]], bufs[b], gsems[b])

        plsc.subcore_barrier()

        hchunks = nchunk // 2
        for h in range(2):
            pltpu.sync_copy(packp_hbm.at[s].at[pl.ds(h * hchunks, hchunks)],
                            pidx)
            for b in range(DEPTH - 1):
                unpack(b, b)
                gather(b)

            @pl.loop(0, hchunks, step=DEPTH)
            def _(j0):
                for b in range(DEPTH):
                    j = j0 + b
                    nxt = (b + DEPTH - 1) % DEPTH

                    @pl.when(j + DEPTH - 1 < hchunks)
                    def _():
                        unpack(j + DEPTH - 1, nxt)
                        gather(nxt)

                    pltpu.make_async_copy(
                        xtab.at[sring.at[b]], bufs[b], gsems[b]).wait()
                    pltpu.sync_copy(bufs[b], acc.at[dring.at[b]], add=True)

        plsc.subcore_barrier()

        @pl.when(s < NS - 1)
        def _():
            pltpu.sync_copy(acc.at[pl.ds(base, rpt)],
                            out_hbm.at[c].at[pl.ds(base, rpt)])

        @pl.when(s == NS - 1)
        def _():
            pltpu.sync_copy(acc.at[pl.ds(base, last_rows)],
                            out_hbm.at[c].at[pl.ds(base, last_rows)])

    return seg(xact, packp)


def _mlp_body(p_ref, x_ref, a_ref, d_ref, b_ref, e_ref, o_ref):
    agg = jnp.concatenate([p_ref[0], p_ref[1]], axis=-1)
    h0 = x_ref[...] + agg
    t = jnp.dot(h0, a_ref[...], preferred_element_type=jnp.float32) + d_ref[...]
    t = jnp.maximum(t, 0.0)
    o = jnp.dot(t, b_ref[...], preferred_element_type=jnp.float32) + e_ref[...]
    o_ref[...] = jnp.maximum(o, 0.0)


def _mlp(p, x, a, dvec, b, evec, blk):
    n, d = x.shape
    d2 = d // 2
    grid = (n // blk,)
    return pl.pallas_call(
        _mlp_body,
        grid=grid,
        in_specs=[
            pl.BlockSpec((NC, blk, d2), lambda i: (0, i, 0)),
            pl.BlockSpec((blk, d), lambda i: (i, 0)),
            pl.BlockSpec((d, d), lambda i: (0, 0)),
            pl.BlockSpec((1, d), lambda i: (0, 0)),
            pl.BlockSpec((d, d), lambda i: (0, 0)),
            pl.BlockSpec((1, d), lambda i: (0, 0)),
        ],
        out_specs=pl.BlockSpec((blk, d), lambda i: (i, 0)),
        out_shape=jax.ShapeDtypeStruct((n, d), jnp.float32),
    )(p, x, a, dvec.reshape(1, d), b, evec.reshape(1, d))


def kernel(x, edge_index, batch, W1a, b1a, g1, be1, rm1, rv1, W1b, b1b,
           W2a, b2a, g2, be2, rm2, rv2, W2b, b2b):
    n, d = x.shape
    d2 = d // 2
    e = edge_index.shape[1]

    per_tile = -(-e // NS)
    nchunk = -(-per_tile // (C * DEPTH * 2)) * DEPTH * 2
    ep = NS * nchunk * C
    src = edge_index[0]
    dst = edge_index[1]
    shift = max(int(n - 1).bit_length(), 1)
    packed = src | (dst << shift)
    packp = jnp.concatenate(
        [packed, jnp.full((ep - e,), n << shift, jnp.int32)]
    ).reshape(NS, nchunk, C)

    inv1 = g1 * lax.rsqrt(rv1 + 1e-5)
    a1 = W1a.T * inv1[None, :]
    d1 = (b1a - rm1) * inv1 + be1
    inv2 = g2 * lax.rsqrt(rv2 + 1e-5)
    a2 = W2a.T * inv2[None, :]
    d2v = (b2a - rm2) * inv2 + be2

    blk = 1000
    p1 = _seg_sum_halves(x, packp, n, nchunk, shift)
    x1 = _mlp(p1, x, a1, d1, W1b.T, b1b, blk)
    p2 = _seg_sum_halves(x1, packp, n, nchunk, shift)
    x2 = _mlp(p2, x1, a2, d2v, W2b.T, b2b, blk)
    return (x1, x2)


# speedup vs baseline: 1.0325x; 1.0325x over previous
import functools

import jax
import jax.numpy as jnp
from jax import lax
from jax.experimental import pallas as pl
from jax.experimental.pallas import tpu as pltpu
from jax.experimental.pallas import tpu_sc as plsc

NC = 2
NS = 16
C = 128
DEPTH = 2


def _seg_sum_halves(xact, packp, n, nchunk, shift):
    d2 = xact.shape[1] // 2
    rpt = -(-(-(-n // NS)) // 8) * 8
    last_rows = n - (NS - 1) * rpt
    last_zero = (n + 8) - (NS - 1) * rpt
    mesh = plsc.VectorSubcoreMesh(core_axis_name="c", subcore_axis_name="s")

    @functools.partial(
        pl.kernel,
        out_type=jax.ShapeDtypeStruct((NC, n, d2), jnp.float32),
        mesh=mesh,
        compiler_params=pltpu.CompilerParams(use_tc_tiling_on_sc=False),
        scratch_types=[
            pltpu.VMEM((nchunk, C), jnp.int32),
            pltpu.VMEM((DEPTH, C), jnp.int32),
            pltpu.VMEM((DEPTH, C), jnp.int32),
            [pltpu.VMEM((C, d2), jnp.float32) for _ in range(DEPTH)],
            [pltpu.SemaphoreType.DMA for _ in range(DEPTH)],
            [pltpu.SemaphoreType.DMA for _ in range(DEPTH)],
            pltpu.VMEM_SHARED((n + 8, d2), jnp.float32),
            pltpu.VMEM_SHARED((n, d2), jnp.float32),
        ],
    )
    def seg(xact_hbm, packp_hbm, out_hbm,
            pidx, sring, dring, bufs, gsems, ssems, acc, xtab):
        c = lax.axis_index("c")
        s = lax.axis_index("s")

        zero = jnp.zeros((16,), jnp.float32)

        @pl.loop(0, C)
        def _(r):
            for col in range(d2 // 16):
                bufs[0][r, pl.ds(col * 16, 16)] = zero

        base = s * rpt

        def zero_acc(cnt):
            nfull, tail = cnt // C, cnt % C
            for k in range(nfull):
                pltpu.sync_copy(bufs[0], acc.at[pl.ds(base + k * C, C)])
            if tail:
                pltpu.sync_copy(bufs[0].at[pl.ds(0, tail)],
                                acc.at[pl.ds(base + nfull * C, tail)])

        @pl.when(s < NS - 1)
        def _():
            zero_acc(rpt)

        @pl.when(s == NS - 1)
        def _():
            zero_acc(last_zero)

        col = pl.ds(c * d2, d2)

        @pl.when(s < NS - 1)
        def _():
            pltpu.sync_copy(xact_hbm.at[pl.ds(base, rpt), col],
                            xtab.at[pl.ds(base, rpt)])

        @pl.when(s == NS - 1)
        def _():
            pltpu.sync_copy(xact_hbm.at[pl.ds(base, last_rows), col],
                            xtab.at[pl.ds(base, last_rows)])

        pltpu.sync_copy(packp_hbm.at[s], pidx)

        mask = jnp.full((16,), (1 << shift) - 1, jnp.int32)

        def unpack(j, slot):
            for col in range(C // 16):
                sl = pl.ds(col * 16, 16)
                v = pidx[j, sl]
                dring[slot, sl] = lax.shift_right_logical(v, shift)
                sring[slot, sl] = lax.bitwise_and(v, mask)

        def gather(b):
            pltpu.async_copy(xtab.at[sring.at[b]], bufs[b], gsems[b])

        plsc.subcore_barrier()

        unpack(0, 0)
        gather(0)

        @pl.loop(0, nchunk, step=DEPTH)
        def _(j0):
            for b in range(DEPTH):
                j = j0 + b

                @pl.when(j + 1 < nchunk)
                def _():
                    @pl.when(j >= 1)
                    def _():
                        pltpu.make_async_copy(
                            bufs[1 - b], acc.at[dring.at[1 - b]],
                            ssems[1 - b]).wait()

                    unpack(j + 1, 1 - b)
                    gather(1 - b)

                pltpu.make_async_copy(
                    xtab.at[sring.at[b]], bufs[b], gsems[b]).wait()
                pltpu.async_copy(bufs[b], acc.at[dring.at[b]], ssems[b],
                                 add=True)

        for b in range(DEPTH):
            pltpu.make_async_copy(
                bufs[b], acc.at[dring.at[b]], ssems[b]).wait()

        plsc.subcore_barrier()

        @pl.when(s < NS - 1)
        def _():
            pltpu.sync_copy(acc.at[pl.ds(base, rpt)],
                            out_hbm.at[c].at[pl.ds(base, rpt)])

        @pl.when(s == NS - 1)
        def _():
            pltpu.sync_copy(acc.at[pl.ds(base, last_rows)],
                            out_hbm.at[c].at[pl.ds(base, last_rows)])

    return seg(xact, packp)


def _mlp_body(p_ref, x_ref, a_ref, d_ref, b_ref, e_ref, o_ref):
    agg = jnp.concatenate([p_ref[0], p_ref[1]], axis=-1)
    h0 = x_ref[...] + agg
    t = jnp.dot(h0, a_ref[...], preferred_element_type=jnp.float32) + d_ref[...]
    t = jnp.maximum(t, 0.0)
    o = jnp.dot(t, b_ref[...], preferred_element_type=jnp.float32) + e_ref[...]
    o_ref[...] = jnp.maximum(o, 0.0)


def _mlp(p, x, a, dvec, b, evec, blk):
    n, d = x.shape
    d2 = d // 2
    grid = (n // blk,)
    return pl.pallas_call(
        _mlp_body,
        grid=grid,
        in_specs=[
            pl.BlockSpec((NC, blk, d2), lambda i: (0, i, 0)),
            pl.BlockSpec((blk, d), lambda i: (i, 0)),
            pl.BlockSpec((d, d), lambda i: (0, 0)),
            pl.BlockSpec((1, d), lambda i: (0, 0)),
            pl.BlockSpec((d, d), lambda i: (0, 0)),
            pl.BlockSpec((1, d), lambda i: (0, 0)),
        ],
        out_specs=pl.BlockSpec((blk, d), lambda i: (i, 0)),
        out_shape=jax.ShapeDtypeStruct((n, d), jnp.float32),
    )(p, x, a, dvec.reshape(1, d), b, evec.reshape(1, d))


def kernel(x, edge_index, batch, W1a, b1a, g1, be1, rm1, rv1, W1b, b1b,
           W2a, b2a, g2, be2, rm2, rv2, W2b, b2b):
    n, d = x.shape
    d2 = d // 2
    e = edge_index.shape[1]

    per_tile = -(-e // NS)
    nchunk = -(-per_tile // (C * DEPTH)) * DEPTH
    ep = NS * nchunk * C
    src = edge_index[0]
    dst = edge_index[1]
    shift = max(int(n - 1).bit_length(), 1)
    packed = src | (dst << shift)
    packp = jnp.concatenate(
        [packed, jnp.full((ep - e,), n << shift, jnp.int32)]
    ).reshape(NS, nchunk, C)

    inv1 = g1 * lax.rsqrt(rv1 + 1e-5)
    a1 = W1a.T * inv1[None, :]
    d1 = (b1a - rm1) * inv1 + be1
    inv2 = g2 * lax.rsqrt(rv2 + 1e-5)
    a2 = W2a.T * inv2[None, :]
    d2v = (b2a - rm2) * inv2 + be2

    blk = 1000
    p1 = _seg_sum_halves(x, packp, n, nchunk, shift)
    x1 = _mlp(p1, x, a1, d1, W1b.T, b1b, blk)
    p2 = _seg_sum_halves(x1, packp, n, nchunk, shift)
    x2 = _mlp(p2, x1, a2, d2v, W2b.T, b2b, blk)
    return (x1, x2)
